# baseline (speedup 1.0000x reference)
import functools

import jax
import jax.numpy as jnp
from jax import lax
from jax.experimental import pallas as pl
from jax.experimental.pallas import tpu as pltpu
from jax.experimental.pallas import tpu_sc as plsc

D = 128
B = 1024
T = 200
ROWS = B * T
NC = 2
NS = 16
L = 16
NW = NC * NS
RPW = ROWS // NW
CH = 200
G = 100
NG = CH // G
NCH = RPW // CH


NB = 3


def _body(idx_hbm, tok_hbm, pos_hbm, out_hbm, idx_v, pos_v, buf, semg, sems):
  wid = lax.axis_index("s") * NC + lax.axis_index("c")
  pltpu.sync_copy(idx_hbm.at[pl.ds(wid * RPW, RPW)], idx_v)
  pltpu.sync_copy(pos_hbm, pos_v)

  def gather_args(j, g):
    b = j % NB
    return (tok_hbm.at[idx_v.at[pl.ds(j * CH, CH)]], buf.at[b], semg.at[b])

  def scatter_args(j):
    b = j % NB
    return (buf.at[b], out_hbm.at[pl.ds(wid * RPW + j * CH, CH)], sems.at[b])

  def start_gather(j):
    pltpu.async_copy(*gather_args(j, 0))

  def wait_gather(j):
    pltpu.make_async_copy(*gather_args(j, 0)).wait()

  start_gather(0)
  start_gather(1)

  def chunk_body(j, carry):
    b = j % NB
    wait_gather(j)

    @plsc.parallel_loop(0, CH, step=1, unroll=2)
    def _add_row(r):
      vals = [pos_v[r, pl.ds(c * L, L)] for c in range(D // L)]
      for c in range(D // L):
        plsc.addupdate(buf.at[b, r, pl.ds(c * L, L)], vals[c])
    pltpu.async_copy(*scatter_args(j))

    @pl.when(j + 2 < NCH)
    def _refill():
      @pl.when(j >= 1)
      def _drain():
        pltpu.make_async_copy(*scatter_args(j - 1)).wait()

      start_gather(j + 2)

    return carry

  lax.fori_loop(0, NCH, chunk_body, 0)
  for j in (NCH - 3, NCH - 2, NCH - 1):
    pltpu.make_async_copy(*scatter_args(j)).wait()


_mesh = plsc.VectorSubcoreMesh(core_axis_name="c", subcore_axis_name="s")

_call = functools.partial(
    pl.kernel,
    mesh=_mesh,
    out_type=jax.ShapeDtypeStruct((ROWS, D), jnp.float32),
    scratch_types=[
        pltpu.VMEM((RPW,), jnp.int32),
        pltpu.VMEM((T, D), jnp.float32),
        pltpu.VMEM((NB, CH, D), jnp.float32),
        pltpu.SemaphoreType.DMA((NB,)),
        pltpu.SemaphoreType.DMA((NB,)),
    ],
)(_body)


@jax.jit
def kernel(idx, token_table, pos_table):
  idx2 = idx.reshape(ROWS).astype(jnp.int32)
  out = _call(idx2, token_table, pos_table[:T])
  return out.reshape(B, T, D)


# speedup vs baseline: 1.0113x; 1.0113x over previous
import functools

import jax
import jax.numpy as jnp
from jax import lax
from jax.experimental import pallas as pl
from jax.experimental.pallas import tpu as pltpu
from jax.experimental.pallas import tpu_sc as plsc

D = 128
B = 1024
T = 200
ROWS = B * T
NC = 2
NS = 16
L = 16
NW = NC * NS
RPW = ROWS // NW
CH = 200
G = 100
NG = CH // G
NCH = RPW // CH


NB = 3


def _body(idx_hbm, tok_hbm, pos_hbm, out_hbm, idx_v, pos_v, buf, semg, sems):
  wid = lax.axis_index("s") * NC + lax.axis_index("c")
  pltpu.sync_copy(idx_hbm.at[pl.ds(wid * NCH, NCH)], idx_v)
  pltpu.sync_copy(pos_hbm, pos_v)

  def gather_args(j, g):
    b = j % NB
    return (tok_hbm.at[idx_v.at[j, g]], buf.at[b, pl.ds(g * G, G)],
            semg.at[b])

  def scatter_args(j):
    b = j % NB
    return (buf.at[b], out_hbm.at[pl.ds(wid * RPW + j * CH, CH)], sems.at[b])

  def start_gather(j):
    for g in range(NG):
      pltpu.async_copy(*gather_args(j, g))

  def wait_gather(j):
    for g in range(NG):
      pltpu.make_async_copy(*gather_args(j, g)).wait()

  start_gather(0)
  start_gather(1)

  def chunk_body(j, carry):
    b = j % NB
    wait_gather(j)

    @plsc.parallel_loop(0, CH, step=1, unroll=2)
    def _add_row(r):
      vals = [pos_v[r, pl.ds(c * L, L)] for c in range(D // L)]
      for c in range(D // L):
        plsc.addupdate(buf.at[b, r, pl.ds(c * L, L)], vals[c])
    pltpu.async_copy(*scatter_args(j))

    @pl.when(j + 2 < NCH)
    def _refill():
      @pl.when(j >= 1)
      def _drain():
        pltpu.make_async_copy(*scatter_args(j - 1)).wait()

      start_gather(j + 2)

    return carry

  lax.fori_loop(0, NCH, chunk_body, 0)
  for j in (NCH - 3, NCH - 2, NCH - 1):
    pltpu.make_async_copy(*scatter_args(j)).wait()


_mesh = plsc.VectorSubcoreMesh(core_axis_name="c", subcore_axis_name="s")

_call = functools.partial(
    pl.kernel,
    mesh=_mesh,
    out_type=jax.ShapeDtypeStruct((ROWS, D), jnp.float32),
    scratch_types=[
        pltpu.VMEM((NCH, NG, G), jnp.int32),
        pltpu.VMEM((T, D), jnp.float32),
        pltpu.VMEM((NB, CH, D), jnp.float32),
        pltpu.SemaphoreType.DMA((NB,)),
        pltpu.SemaphoreType.DMA((NB,)),
    ],
)(_body)


@jax.jit
def kernel(idx, token_table, pos_table):
  idx2 = idx.reshape(NW * NCH, NG, G).astype(jnp.int32)
  out = _call(idx2, token_table, pos_table[:T])
  return out.reshape(B, T, D)
